# baseline (speedup 1.0000x reference)
import jax
import jax.numpy as jnp
from jax import lax
from jax.experimental import pallas as pl
from jax.experimental.pallas import tpu as pltpu
from jax.experimental.pallas import tpu_sc as plsc

SIZE = 1000000
DIM = 64
BATCH = 16384
L = 16
NW = 32
RPW = SIZE // NW
NVR = BATCH // L
CW = 128
NCH = BATCH // CW
CCH = 125
NCP = RPW // (2 * CCH)

_mesh = plsc.VectorSubcoreMesh(core_axis_name="c", subcore_axis_name="s")


def _body(bank_hbm, idx_hbm, data_hbm, out_hbm, idxbuf, stamp, idxf, jf,
          rows, cbuf, idxsem, i0sem, i1sem, o0sem, o1sem, gsem, ssem):
    wid = lax.axis_index("s") * 2 + lax.axis_index("c")
    base = wid * RPW

    idx_cp = pltpu.async_copy(idx_hbm, idxbuf, idxsem)

    def cp(kk, carry):
        r0 = base + (2 * kk) * CCH
        r1 = r0 + CCH
        c0 = pltpu.async_copy(bank_hbm.at[pl.ds(r0, CCH)], cbuf.at[0], i0sem)
        c1 = pltpu.async_copy(bank_hbm.at[pl.ds(r1, CCH)], cbuf.at[1], i1sem)
        c0.wait()
        o0 = pltpu.async_copy(cbuf.at[0], out_hbm.at[pl.ds(r0, CCH)], o0sem)
        c1.wait()
        o1 = pltpu.async_copy(cbuf.at[1], out_hbm.at[pl.ds(r1, CCH)], o1sem)
        o0.wait()
        o1.wait()
        return carry

    lax.fori_loop(0, NCP, cp, jnp.int32(0))

    idx_cp.wait()
    iota = lax.iota(jnp.int32, L)
    zero = jnp.zeros((L,), jnp.int32)

    def p1(v, carry):
        ids = idxbuf[pl.ds(v * L, L)]
        jv = v * L + iota
        m = (ids >= base) & (ids < base + RPW)
        rloc = jnp.clip(ids - base, 0, RPW - 1)
        plsc.store_scatter(stamp, [rloc], jv, mask=m)
        return carry

    lax.fori_loop(0, NVR, p1, jnp.int32(0))

    def p2(v, carry):
        n, lastid, lastj = carry
        ids = idxbuf[pl.ds(v * L, L)]
        jv = v * L + iota
        m = (ids >= base) & (ids < base + RPW)
        rloc = jnp.clip(ids - base, 0, RPW - 1)
        w = plsc.load_gather(stamp, [rloc], mask=m)
        keep = m & (w == jv)
        ki = keep.astype(jnp.int32)
        cs = plsc.cumsum(ki)
        cnt = jnp.sum(ki)
        pos = n + cs - 1
        posc = jnp.clip(pos, 0, BATCH - 1)
        row = jnp.right_shift(posc, 7)
        col = posc & (CW - 1)
        plsc.store_scatter(idxf, [row, col], ids, mask=keep)
        plsc.store_scatter(jf, [row, col], jv, mask=keep)
        nn = n + cnt
        sel = keep & (pos == nn - 1)
        lid = jnp.sum(jnp.where(sel, ids, zero))
        lj = jnp.sum(jnp.where(sel, jv, zero))
        has = cnt > 0
        return (nn, jnp.where(has, lid, lastid), jnp.where(has, lj, lastj))

    n, lastid, lastj = lax.fori_loop(
        0, NVR, p2, (jnp.int32(0), jnp.int32(0), jnp.int32(0)))

    ntot = jnp.bitwise_and(n + (CW - 1), -CW)
    lid_v = zero + lastid
    lj_v = zero + lastj
    for p in range(CW // L):
        lanepos = n + p * L + iota
        mp = lanepos < ntot
        pc = jnp.clip(lanepos, 0, BATCH - 1)
        row = jnp.right_shift(pc, 7)
        col = pc & (CW - 1)
        plsc.store_scatter(idxf, [row, col], lid_v, mask=mp)
        plsc.store_scatter(jf, [row, col], lj_v, mask=mp)

    nch = jnp.right_shift(ntot, 7)

    def dma(ch, carry):
        pltpu.async_copy(data_hbm.at[jf.at[ch]], rows, gsem).wait()
        pltpu.async_copy(rows, out_hbm.at[idxf.at[ch]], ssem).wait()
        return carry

    lax.fori_loop(0, nch, dma, jnp.int32(0))


_memory_bank_update = pl.kernel(
    _body,
    out_type=jax.ShapeDtypeStruct((SIZE, DIM), jnp.float32),
    mesh=_mesh,
    compiler_params=pltpu.CompilerParams(
        needs_layout_passes=False, use_tc_tiling_on_sc=False),
    scratch_types=[
        pltpu.VMEM((BATCH,), jnp.int32),
        pltpu.VMEM((RPW,), jnp.int32),
        pltpu.VMEM((NCH, CW), jnp.int32),
        pltpu.VMEM((NCH, CW), jnp.int32),
        pltpu.VMEM((CW, DIM), jnp.float32),
        pltpu.VMEM((2, CCH, DIM), jnp.float32),
        pltpu.SemaphoreType.DMA,
        pltpu.SemaphoreType.DMA,
        pltpu.SemaphoreType.DMA,
        pltpu.SemaphoreType.DMA,
        pltpu.SemaphoreType.DMA,
        pltpu.SemaphoreType.DMA,
        pltpu.SemaphoreType.DMA,
    ],
)


def kernel(bank, indices, data_memory):
    return _memory_bank_update(bank, indices.astype(jnp.int32), data_memory)


# speedup vs baseline: 1.0482x; 1.0482x over previous
import jax
import jax.numpy as jnp
from jax import lax
from jax.experimental import pallas as pl
from jax.experimental.pallas import tpu as pltpu
from jax.experimental.pallas import tpu_sc as plsc

SIZE = 1000000
DIM = 64
BATCH = 16384
L = 16
NW = 32
RPW = SIZE // NW
NVR = BATCH // L
CW = 128
NCH = BATCH // CW
CCH = 125
NCC = RPW // CCH
NBUF = 4

_mesh = plsc.VectorSubcoreMesh(core_axis_name="c", subcore_axis_name="s")


def _body(bank_hbm, idx_hbm, data_hbm, out_hbm, idxbuf, stamp, idxf, jf,
          rows, cbuf, idxsem, isems, osems, gsem, ssem):
    wid = lax.axis_index("s") * 2 + lax.axis_index("c")
    base = wid * RPW

    idx_cp = pltpu.async_copy(idx_hbm, idxbuf, idxsem)

    def _in_start(k):
        b = k & (NBUF - 1)
        pltpu.async_copy(
            bank_hbm.at[pl.ds(base + k * CCH, CCH)], cbuf.at[b], isems.at[b])

    def _in_wait(k):
        b = k & (NBUF - 1)
        pltpu.make_async_copy(
            bank_hbm.at[pl.ds(base + k * CCH, CCH)], cbuf.at[b],
            isems.at[b]).wait()

    def _out_start(k):
        b = k & (NBUF - 1)
        pltpu.async_copy(
            cbuf.at[b], out_hbm.at[pl.ds(base + k * CCH, CCH)], osems.at[b])

    def _out_wait(k):
        b = k & (NBUF - 1)
        pltpu.make_async_copy(
            cbuf.at[b], out_hbm.at[pl.ds(base + k * CCH, CCH)],
            osems.at[b]).wait()

    def cp(k, carry):
        @pl.when(k >= NBUF)
        def _():
            _out_wait(k - NBUF)

        _in_start(k)

        @pl.when(k >= 2)
        def _():
            _in_wait(k - 2)
            _out_start(k - 2)

        return carry

    lax.fori_loop(0, NCC, cp, jnp.int32(0))
    for k in range(NCC - 2, NCC):
        _in_wait(k)
        _out_start(k)
    for k in range(NCC - NBUF, NCC):
        _out_wait(k)

    idx_cp.wait()
    iota = lax.iota(jnp.int32, L)
    zero = jnp.zeros((L,), jnp.int32)

    def p1(v, carry):
        ids = idxbuf[pl.ds(v * L, L)]
        jv = v * L + iota
        m = (ids >= base) & (ids < base + RPW)
        rloc = jnp.clip(ids - base, 0, RPW - 1)
        plsc.store_scatter(stamp, [rloc], jv, mask=m)
        return carry

    lax.fori_loop(0, NVR, p1, jnp.int32(0))

    def p2(v, carry):
        n, lastid, lastj = carry
        ids = idxbuf[pl.ds(v * L, L)]
        jv = v * L + iota
        m = (ids >= base) & (ids < base + RPW)
        rloc = jnp.clip(ids - base, 0, RPW - 1)
        w = plsc.load_gather(stamp, [rloc], mask=m)
        keep = m & (w == jv)
        ki = keep.astype(jnp.int32)
        cs = plsc.cumsum(ki)
        cnt = jnp.sum(ki)
        pos = n + cs - 1
        posc = jnp.clip(pos, 0, BATCH - 1)
        row = jnp.right_shift(posc, 7)
        col = posc & (CW - 1)
        plsc.store_scatter(idxf, [row, col], ids, mask=keep)
        plsc.store_scatter(jf, [row, col], jv, mask=keep)
        nn = n + cnt
        sel = keep & (pos == nn - 1)
        lid = jnp.sum(jnp.where(sel, ids, zero))
        lj = jnp.sum(jnp.where(sel, jv, zero))
        has = cnt > 0
        return (nn, jnp.where(has, lid, lastid), jnp.where(has, lj, lastj))

    n, lastid, lastj = lax.fori_loop(
        0, NVR, p2, (jnp.int32(0), jnp.int32(0), jnp.int32(0)))

    ntot = jnp.bitwise_and(n + (CW - 1), -CW)
    lid_v = zero + lastid
    lj_v = zero + lastj
    for p in range(CW // L):
        lanepos = n + p * L + iota
        mp = lanepos < ntot
        pc = jnp.clip(lanepos, 0, BATCH - 1)
        row = jnp.right_shift(pc, 7)
        col = pc & (CW - 1)
        plsc.store_scatter(idxf, [row, col], lid_v, mask=mp)
        plsc.store_scatter(jf, [row, col], lj_v, mask=mp)

    nch = jnp.right_shift(ntot, 7)

    def dma(ch, carry):
        pltpu.async_copy(data_hbm.at[jf.at[ch]], rows, gsem).wait()
        pltpu.async_copy(rows, out_hbm.at[idxf.at[ch]], ssem).wait()
        return carry

    lax.fori_loop(0, nch, dma, jnp.int32(0))


_memory_bank_update = pl.kernel(
    _body,
    out_type=jax.ShapeDtypeStruct((SIZE, DIM), jnp.float32),
    mesh=_mesh,
    compiler_params=pltpu.CompilerParams(
        needs_layout_passes=False, use_tc_tiling_on_sc=False),
    scratch_types=[
        pltpu.VMEM((BATCH,), jnp.int32),
        pltpu.VMEM((RPW,), jnp.int32),
        pltpu.VMEM((NCH, CW), jnp.int32),
        pltpu.VMEM((NCH, CW), jnp.int32),
        pltpu.VMEM((CW, DIM), jnp.float32),
        pltpu.VMEM((NBUF, CCH, DIM), jnp.float32),
        pltpu.SemaphoreType.DMA,
        pltpu.SemaphoreType.DMA((NBUF,)),
        pltpu.SemaphoreType.DMA((NBUF,)),
        pltpu.SemaphoreType.DMA,
        pltpu.SemaphoreType.DMA,
    ],
)


def kernel(bank, indices, data_memory):
    return _memory_bank_update(bank, indices.astype(jnp.int32), data_memory)


# speedup vs baseline: 1.0700x; 1.0208x over previous
import jax
import jax.numpy as jnp
from jax import lax
from jax.experimental import pallas as pl
from jax.experimental.pallas import tpu as pltpu
from jax.experimental.pallas import tpu_sc as plsc

SIZE = 1000000
DIM = 64
BATCH = 16384
L = 16
NW = 32
RPW = SIZE // NW
NVR = BATCH // L
CW = 128
NCH = BATCH // CW
CCH = 125
NCC = RPW // CCH
NBUF = 4
HALF = NCC // 2
VP = -(-NVR // HALF)

_mesh = plsc.VectorSubcoreMesh(core_axis_name="c", subcore_axis_name="s")


def _body(bank_hbm, idx_hbm, data_hbm, out_hbm, idxbuf, stamp, idxf, jf,
          rows, cbuf, idxsem, isems, osems, gsem, ssem):
    wid = lax.axis_index("s") * 2 + lax.axis_index("c")
    base = wid * RPW

    pltpu.async_copy(idx_hbm, idxbuf, idxsem).wait()
    iota = lax.iota(jnp.int32, L)
    zero = jnp.zeros((L,), jnp.int32)

    def _in_start(k):
        b = k & (NBUF - 1)
        pltpu.async_copy(
            bank_hbm.at[pl.ds(base + k * CCH, CCH)], cbuf.at[b], isems.at[b])

    def _in_wait(k):
        b = k & (NBUF - 1)
        pltpu.make_async_copy(
            bank_hbm.at[pl.ds(base + k * CCH, CCH)], cbuf.at[b],
            isems.at[b]).wait()

    def _out_start(k):
        b = k & (NBUF - 1)
        pltpu.async_copy(
            cbuf.at[b], out_hbm.at[pl.ds(base + k * CCH, CCH)], osems.at[b])

    def _out_wait(k):
        b = k & (NBUF - 1)
        pltpu.make_async_copy(
            cbuf.at[b], out_hbm.at[pl.ds(base + k * CCH, CCH)],
            osems.at[b]).wait()

    def _copy_step(k):
        @pl.when(k >= NBUF)
        def _():
            _out_wait(k - NBUF)

        _in_start(k)

        @pl.when(k >= 2)
        def _():
            _in_wait(k - 2)
            _out_start(k - 2)

    def p1(v):
        vc = jnp.minimum(v, NVR - 1)
        ids = idxbuf[pl.ds(vc * L, L)]
        jv = vc * L + iota
        m = (ids >= base) & (ids < base + RPW) & (v < NVR)
        rloc = jnp.clip(ids - base, 0, RPW - 1)
        plsc.store_scatter(stamp, [rloc], jv, mask=m)

    def cpA(k, carry):
        _copy_step(k)
        for q in range(VP):
            p1(k * VP + q)
        return carry

    lax.fori_loop(0, HALF, cpA, jnp.int32(0))

    def p2(v, carry):
        n, lastid, lastj = carry
        vc = jnp.minimum(v, NVR - 1)
        ids = idxbuf[pl.ds(vc * L, L)]
        jv = vc * L + iota
        m = (ids >= base) & (ids < base + RPW) & (v < NVR)
        rloc = jnp.clip(ids - base, 0, RPW - 1)
        w = plsc.load_gather(stamp, [rloc], mask=m)
        keep = m & (w == jv)
        ki = keep.astype(jnp.int32)
        cs = plsc.cumsum(ki)
        cnt = jnp.sum(ki)
        pos = n + cs - 1
        posc = jnp.clip(pos, 0, BATCH - 1)
        row = jnp.right_shift(posc, 7)
        col = posc & (CW - 1)
        plsc.store_scatter(idxf, [row, col], ids, mask=keep)
        plsc.store_scatter(jf, [row, col], jv, mask=keep)
        nn = n + cnt
        sel = keep & (pos == nn - 1)
        lid = jnp.sum(jnp.where(sel, ids, zero))
        lj = jnp.sum(jnp.where(sel, jv, zero))
        has = cnt > 0
        return (nn, jnp.where(has, lid, lastid), jnp.where(has, lj, lastj))

    def cpB(k, carry):
        _copy_step(k)
        for q in range(VP):
            carry = p2((k - HALF) * VP + q, carry)
        return carry

    n, lastid, lastj = lax.fori_loop(
        HALF, NCC, cpB, (jnp.int32(0), jnp.int32(0), jnp.int32(0)))

    for k in range(NCC - 2, NCC):
        _in_wait(k)
        _out_start(k)
    for k in range(NCC - NBUF, NCC):
        _out_wait(k)

    ntot = jnp.bitwise_and(n + (CW - 1), -CW)
    lid_v = zero + lastid
    lj_v = zero + lastj
    for p in range(CW // L):
        lanepos = n + p * L + iota
        mp = lanepos < ntot
        pc = jnp.clip(lanepos, 0, BATCH - 1)
        row = jnp.right_shift(pc, 7)
        col = pc & (CW - 1)
        plsc.store_scatter(idxf, [row, col], lid_v, mask=mp)
        plsc.store_scatter(jf, [row, col], lj_v, mask=mp)

    nch = jnp.right_shift(ntot, 7)

    def dma(ch, carry):
        pltpu.async_copy(data_hbm.at[jf.at[ch]], rows, gsem).wait()
        pltpu.async_copy(rows, out_hbm.at[idxf.at[ch]], ssem).wait()
        return carry

    lax.fori_loop(0, nch, dma, jnp.int32(0))


_memory_bank_update = pl.kernel(
    _body,
    out_type=jax.ShapeDtypeStruct((SIZE, DIM), jnp.float32),
    mesh=_mesh,
    compiler_params=pltpu.CompilerParams(
        needs_layout_passes=False, use_tc_tiling_on_sc=False),
    scratch_types=[
        pltpu.VMEM((BATCH,), jnp.int32),
        pltpu.VMEM((RPW,), jnp.int32),
        pltpu.VMEM((NCH, CW), jnp.int32),
        pltpu.VMEM((NCH, CW), jnp.int32),
        pltpu.VMEM((CW, DIM), jnp.float32),
        pltpu.VMEM((NBUF, CCH, DIM), jnp.float32),
        pltpu.SemaphoreType.DMA,
        pltpu.SemaphoreType.DMA((NBUF,)),
        pltpu.SemaphoreType.DMA((NBUF,)),
        pltpu.SemaphoreType.DMA,
        pltpu.SemaphoreType.DMA,
    ],
)


def kernel(bank, indices, data_memory):
    return _memory_bank_update(bank, indices.astype(jnp.int32), data_memory)
